# Initial kernel scaffold; baseline (speedup 1.0000x reference)
import jax
import jax.numpy as jnp
from jax.experimental import pallas as pl


def kernel(node_features, edge_radial, edge_angular, edge_index, mlp1_w, mlp1_b, mlp2_w0, mlp2_b0, mlp2_w1, mlp2_b1, mlp2_w2, mlp2_b2, mlp2_w3, mlp2_b3, self_w, self_b, neigh_w, neigh_b, emlp1_w, emlp1_b, emlp2_w0, emlp2_b0, emlp2_w1, emlp2_b1, emlp2_w2, emlp2_b2, emlp2_w3, emlp2_b3):
    raise NotImplementedError("write your pallas kernel here")


# speedup vs baseline: 1.4939x; 1.4939x over previous
import functools

import jax
import jax.numpy as jnp
from jax import lax
from jax.experimental import pallas as pl
from jax.experimental.pallas import tpu as pltpu
from jax.experimental.pallas import tpu_sc as plsc

N = 10000
E = 160000
ND = 256
ERD = 16
EAD = 16
ACD = ND + ERD + EAD

NC = 2
NS = 16
NW = NC * NS


_EPW = E // NW
_ACH = 200
_ANCH = _EPW // _ACH

_vmesh = plsc.VectorSubcoreMesh(core_axis_name="c", subcore_axis_name="s")


@functools.partial(
    pl.kernel,
    mesh=_vmesh,
    out_type=jax.ShapeDtypeStruct((E, ND), jnp.float32),
    scratch_types=[
        pltpu.VMEM((_ACH,), jnp.int32),
        pltpu.VMEM((_ACH,), jnp.int32),
        pltpu.VMEM((_ACH, ND), jnp.float32),
    ],
)
def _sc_gather_compose(nf_hbm, src_hbm, x2_hbm, srcv, src2v, rows):
    wid = lax.axis_index("s") * NC + lax.axis_index("c")

    @pl.loop(0, _ANCH)
    def _(k):
        base = wid * _EPW + k * _ACH
        pltpu.sync_copy(src_hbm.at[pl.ds(base, _ACH)], srcv)
        pltpu.sync_copy(src_hbm.at[srcv], src2v)
        pltpu.sync_copy(nf_hbm.at[src2v], rows)
        pltpu.sync_copy(rows, x2_hbm.at[pl.ds(base, _ACH)])


_HF = ACD // 2
_NRPS = 626
_NPAD = NS * _NRPS
_CCH = 200
_EPS = E // NS
_CNCH = _EPS // _CCH


@functools.partial(
    pl.kernel,
    mesh=_vmesh,
    out_type=[
        jax.ShapeDtypeStruct((_NPAD, _HF), jnp.float32),
        jax.ShapeDtypeStruct((_NPAD, _HF), jnp.float32),
    ],
    scratch_types=[
        pltpu.VMEM((_CCH,), jnp.int32),
        pltpu.VMEM((_CCH, _HF), jnp.float32),
        pltpu.VMEM_SHARED((_NPAD, _HF), jnp.float32),
    ],
    compiler_params=pltpu.CompilerParams(use_tc_tiling_on_sc=False),
)
def _sc_scatter_add(sub_hbm, dst_hbm, zeros_hbm, m0_hbm, m1_hbm,
                    dstv, rows, acc):
    cid = lax.axis_index("c")
    sid = lax.axis_index("s")

    pltpu.sync_copy(zeros_hbm, acc.at[pl.ds(sid * _NRPS, _NRPS)])
    plsc.subcore_barrier()

    @pl.loop(0, _CNCH)
    def _(k):
        base = sid * _EPS + k * _CCH
        pltpu.sync_copy(dst_hbm.at[pl.ds(base, _CCH)], dstv)

        @pl.when(cid == 0)
        def _():
            pltpu.sync_copy(sub_hbm.at[0, pl.ds(base, _CCH)], rows)

        @pl.when(cid == 1)
        def _():
            pltpu.sync_copy(sub_hbm.at[1, pl.ds(base, _CCH)], rows)

        pltpu.sync_copy(rows, acc.at[dstv], add=True)

    plsc.subcore_barrier()
    out_rows = pl.ds(sid * _NRPS, _NRPS)

    @pl.when(cid == 0)
    def _():
        pltpu.sync_copy(acc.at[out_rows], m0_hbm.at[out_rows])

    @pl.when(cid == 1)
    def _():
        pltpu.sync_copy(acc.at[out_rows], m1_hbm.at[out_rows])


@functools.partial(
    pl.kernel,
    mesh=_vmesh,
    out_type=[
        jax.ShapeDtypeStruct((E, ND), jnp.float32),
        jax.ShapeDtypeStruct((E, ND), jnp.float32),
    ],
    scratch_types=[
        pltpu.VMEM((_ACH,), jnp.int32),
        pltpu.VMEM((_ACH,), jnp.int32),
        pltpu.VMEM((_ACH, ND), jnp.float32),
        pltpu.VMEM((_ACH, ND), jnp.float32),
    ],
)
def _sc_gather_pair(nfu_hbm, src_hbm, dst_hbm, s_hbm, t_hbm,
                    srcv, dstv, rs, rt):
    wid = lax.axis_index("s") * NC + lax.axis_index("c")

    @pl.loop(0, _ANCH)
    def _(k):
        base = wid * _EPW + k * _ACH
        sl = pl.ds(base, _ACH)
        pltpu.sync_copy(src_hbm.at[sl], srcv)
        pltpu.sync_copy(dst_hbm.at[sl], dstv)
        pltpu.sync_copy(nfu_hbm.at[srcv], rs)
        pltpu.sync_copy(nfu_hbm.at[dstv], rt)
        pltpu.sync_copy(rs, s_hbm.at[sl])
        pltpu.sync_copy(rt, t_hbm.at[sl])


_BE = 2000


def _tc_sub_body(x2, er, ea, wa, wb, wc, b, o):
    acc = jnp.dot(x2[...], wa[...], preferred_element_type=jnp.float32)
    acc += jnp.dot(er[...], wb[...], preferred_element_type=jnp.float32)
    acc += jnp.dot(ea[...], wc[...], preferred_element_type=jnp.float32)
    s = jax.nn.sigmoid(acc + b[...])
    o[0] = s[:, :_HF]
    o[1] = s[:, _HF:]


def _tc_sub(x2, er, ea, wa, wb, wc, b):
    g = E // _BE
    return pl.pallas_call(
        _tc_sub_body,
        grid=(g,),
        in_specs=[
            pl.BlockSpec((_BE, ND), lambda i: (i, 0)),
            pl.BlockSpec((_BE, ERD), lambda i: (i, 0)),
            pl.BlockSpec((_BE, EAD), lambda i: (i, 0)),
            pl.BlockSpec((ND, ACD), lambda i: (0, 0)),
            pl.BlockSpec((ERD, ACD), lambda i: (0, 0)),
            pl.BlockSpec((EAD, ACD), lambda i: (0, 0)),
            pl.BlockSpec((1, ACD), lambda i: (0, 0)),
        ],
        out_specs=pl.BlockSpec((2, _BE, _HF), lambda i: (0, i, 0)),
        out_shape=jax.ShapeDtypeStruct((2, E, _HF), jnp.float32),
    )(x2, er, ea, wa, wb, wc, b)


_BN = 1000


def _tc_node_body(m0, m1, nf, w0a, w0b, b0, w1, b1, w2, b2, w3, b3,
                  ws, bs, wn, bn, o):
    h = jnp.dot(m0[...], w0a[...], preferred_element_type=jnp.float32)
    h += jnp.dot(m1[...], w0b[...], preferred_element_type=jnp.float32)
    h = jax.nn.relu(h + b0[...])
    h = jax.nn.relu(jnp.dot(h, w1[...], preferred_element_type=jnp.float32)
                    + b1[...])
    h = jax.nn.relu(jnp.dot(h, w2[...], preferred_element_type=jnp.float32)
                    + b2[...])
    h = jnp.dot(h, w3[...], preferred_element_type=jnp.float32) + b3[...]
    nfv = nf[...]
    z = jnp.dot(nfv, ws[...], preferred_element_type=jnp.float32) + bs[...]
    z += jnp.dot(h, wn[...], preferred_element_type=jnp.float32) + bn[...]
    o[...] = jax.nn.sigmoid(z) + nfv


def _tc_node(m0, m1, nf, w0a, w0b, b0, w1, b1, w2, b2, w3, b3, ws, bs,
             wn, bn):
    g = N // _BN
    full = lambda r, c: pl.BlockSpec((r, c), lambda i: (0, 0))
    return pl.pallas_call(
        _tc_node_body,
        grid=(g,),
        in_specs=[
            pl.BlockSpec((_BN, _HF), lambda i: (i, 0)),
            pl.BlockSpec((_BN, _HF), lambda i: (i, 0)),
            pl.BlockSpec((_BN, ND), lambda i: (i, 0)),
            full(_HF, 176), full(_HF, 176), full(1, 176),
            full(176, 64), full(1, 64),
            full(64, 128), full(1, 128),
            full(128, ND), full(1, ND),
            full(ND, ND), full(1, ND),
            full(ND, ND), full(1, ND),
        ],
        out_specs=pl.BlockSpec((_BN, ND), lambda i: (i, 0)),
        out_shape=jax.ShapeDtypeStruct((N, ND), jnp.float32),
    )(m0, m1, nf, w0a, w0b, b0, w1, b1, w2, b2, w3, b3, ws, bs, wn, bn)


_BF = 2000


def _tc_edge_body(s, t, er, ea, ewa, ewb, ewc, eb, wa, wb, wc, b,
                  v0, c0, v1, c1, v2, c2, v3, c3, o):
    a1 = jnp.dot(s[...], ewa[...], preferred_element_type=jnp.float32)
    a1 += jnp.dot(er[...], ewb[...], preferred_element_type=jnp.float32)
    a1 += jnp.dot(ea[...], ewc[...], preferred_element_type=jnp.float32)
    a2 = jnp.dot(t[...], wa[...], preferred_element_type=jnp.float32)
    a2 += jnp.dot(er[...], wb[...], preferred_element_type=jnp.float32)
    a2 += jnp.dot(ea[...], wc[...], preferred_element_type=jnp.float32)
    g = jax.nn.sigmoid(a1 + eb[...]) + jax.nn.sigmoid(a2 + b[...])
    g = jax.nn.relu(jnp.dot(g, v0[...], preferred_element_type=jnp.float32)
                    + c0[...])
    g = jax.nn.relu(jnp.dot(g, v1[...], preferred_element_type=jnp.float32)
                    + c1[...])
    g = jax.nn.relu(jnp.dot(g, v2[...], preferred_element_type=jnp.float32)
                    + c2[...])
    o[...] = jnp.dot(g, v3[...], preferred_element_type=jnp.float32) + c3[...]


def _tc_edge(s, t, er, ea, ewa, ewb, ewc, eb, wa, wb, wc, b,
             v0, c0, v1, c1, v2, c2, v3, c3):
    g = E // _BF
    full = lambda r, c: pl.BlockSpec((r, c), lambda i: (0, 0))
    return pl.pallas_call(
        _tc_edge_body,
        grid=(g,),
        in_specs=[
            pl.BlockSpec((_BF, ND), lambda i: (i, 0)),
            pl.BlockSpec((_BF, ND), lambda i: (i, 0)),
            pl.BlockSpec((_BF, ERD), lambda i: (i, 0)),
            pl.BlockSpec((_BF, EAD), lambda i: (i, 0)),
            full(ND, ACD), full(ERD, ACD), full(EAD, ACD), full(1, ACD),
            full(ND, ACD), full(ERD, ACD), full(EAD, ACD), full(1, ACD),
            full(ACD, 148), full(1, 148),
            full(148, 8), full(1, 8),
            full(8, 16), full(1, 16),
            full(16, 32), full(1, 32),
        ],
        out_specs=pl.BlockSpec((_BF, 32), lambda i: (i, 0)),
        out_shape=jax.ShapeDtypeStruct((E, 32), jnp.float32),
    )(s, t, er, ea, ewa, ewb, ewc, eb, wa, wb, wc, b,
      v0, c0, v1, c1, v2, c2, v3, c3)


def kernel(node_features, edge_radial, edge_angular, edge_index,
           mlp1_w, mlp1_b, mlp2_w0, mlp2_b0, mlp2_w1, mlp2_b1,
           mlp2_w2, mlp2_b2, mlp2_w3, mlp2_b3, self_w, self_b,
           neigh_w, neigh_b, emlp1_w, emlp1_b, emlp2_w0, emlp2_b0,
           emlp2_w1, emlp2_b1, emlp2_w2, emlp2_b2, emlp2_w3, emlp2_b3):
    src = edge_index[0]
    dst = edge_index[1]

    w1t = mlp1_w.T
    wa, wb, wc = w1t[:ND], w1t[ND:ND + ERD], w1t[ND + ERD:]
    b1r = mlp1_b[None, :]
    e1t = emlp1_w.T
    ewa, ewb, ewc = e1t[:ND], e1t[ND:ND + ERD], e1t[ND + ERD:]
    eb1r = emlp1_b[None, :]

    w0t = mlp2_w0.T
    w0a, w0b = w0t[:_HF], w0t[_HF:]

    x2 = _sc_gather_compose(node_features, src)
    sub = _tc_sub(x2, edge_radial, edge_angular, wa, wb, wc, b1r)

    zeros = jnp.zeros((_NRPS, _HF), jnp.float32)
    m0, m1 = _sc_scatter_add(sub, dst, zeros)

    nfu = _tc_node(m0[:N], m1[:N], node_features,
                   w0a, w0b, mlp2_b0[None, :],
                   mlp2_w1.T, mlp2_b1[None, :],
                   mlp2_w2.T, mlp2_b2[None, :],
                   mlp2_w3.T, mlp2_b3[None, :],
                   self_w.T, self_b[None, :],
                   neigh_w.T, neigh_b[None, :])

    s_rows, t_rows = _sc_gather_pair(nfu, src, dst)

    e = _tc_edge(s_rows, t_rows, edge_radial, edge_angular,
                 ewa, ewb, ewc, eb1r, wa, wb, wc, b1r,
                 emlp2_w0.T, emlp2_b0[None, :],
                 emlp2_w1.T, emlp2_b1[None, :],
                 emlp2_w2.T, emlp2_b2[None, :],
                 emlp2_w3.T, emlp2_b3[None, :])

    return (nfu, e)
